# baseline (speedup 1.0000x reference)
import jax
import jax.numpy as jnp
from jax import lax
from jax.experimental import pallas as pl
from jax.experimental.pallas import tpu as pltpu
from jax.experimental.pallas import tpu_sc as plsc

_B, _L, _V, _H = 16384, 50, 100000, 256
_CLAMP = 127.0

_NC, _NS = 2, 16
_NW = _NC * _NS
_NBAG = 2 * _B
_BPW = _NBAG // _NW
_CBAG = 128
_NCH = _BPW // _CBAG
_NVR = _H // 16
_NSPL = 4


def _bag_body(idx_ref, emb_ref, out_ref, idx_v, buf_v, acc_v, sem0, sem1):
    cid = lax.axis_index("c")
    sid = lax.axis_index("s")
    wid = sid * _NC + cid
    zero = jnp.zeros((16,), jnp.float32)

    def chunk(c, carry):
        pltpu.sync_copy(idx_ref.at[wid * _NCH + c], idx_v)

        @plsc.parallel_loop(0, _CBAG, unroll=2)
        def _(r):
            for j in range(_NVR):
                acc_v[r, pl.ds(16 * j, 16)] = zero

        def accum(k):
            @plsc.parallel_loop(0, _CBAG, unroll=4)
            def _(r):
                for j in range(_NVR):
                    sl = pl.ds(16 * j, 16)
                    plsc.addupdate(acc_v.at[r, sl], buf_v[k, r, sl])

        def gath(l, k, sem):
            for s in range(_NSPL):
                rs = pl.ds(s * (_CBAG // _NSPL), _CBAG // _NSPL)
                pltpu.async_copy(emb_ref.at[idx_v.at[l, rs]],
                                 buf_v.at[k, rs, :], sem)

        def waitg(l, k, sem):
            for s in range(_NSPL):
                rs = pl.ds(s * (_CBAG // _NSPL), _CBAG // _NSPL)
                pltpu.make_async_copy(emb_ref.at[idx_v.at[l, rs]],
                                      buf_v.at[k, rs, :], sem).wait()

        gath(0, 0, sem0)

        def pos(p, cc):
            l0 = 2 * p

            @pl.when(l0 + 1 < _L)
            def _():
                gath(l0 + 1, 1, sem1)

            waitg(l0, 0, sem0)
            accum(0)

            @pl.when(l0 + 2 < _L)
            def _():
                gath(l0 + 2, 0, sem0)

            @pl.when(l0 + 1 < _L)
            def _():
                waitg(l0 + 1, 1, sem1)
                accum(1)

            return cc

        lax.fori_loop(0, (_L + 1) // 2, pos, 0)
        pltpu.sync_copy(acc_v,
                        out_ref.at[pl.ds(wid * _BPW + c * _CBAG, _CBAG), :])
        return carry

    lax.fori_loop(0, _NCH, chunk, 0)


_bag = pl.kernel(
    _bag_body,
    out_type=jax.ShapeDtypeStruct((_NBAG, _H), jnp.float32),
    mesh=plsc.VectorSubcoreMesh(
        core_axis_name="c", subcore_axis_name="s",
        num_cores=_NC, num_subcores=_NS,
    ),
    scratch_types=[
        pltpu.VMEM((_L, _CBAG), jnp.int32),
        pltpu.VMEM((2, _CBAG, _H), jnp.float32),
        pltpu.VMEM((_CBAG, _H), jnp.float32),
        pltpu.SemaphoreType.DMA,
        pltpu.SemaphoreType.DMA,
    ],
)

_BLK = 2048


def _mlp_body(accw_ref, accb_ref, stm_ref, b1_ref, w1_ref, c1_ref,
              w2_ref, c2_ref, w3_ref, c3_ref, y_ref):
    white = stm_ref[...] == 0
    aw = accw_ref[...] + b1_ref[...]
    ab = accb_ref[...] + b1_ref[...]
    astm = jnp.where(white, aw, ab)
    anstm = jnp.where(white, ab, aw)
    x = jnp.clip(jnp.concatenate([astm, anstm], axis=1), 0.0, _CLAMP)
    h = lax.dot_general(x, w1_ref[...], (((1,), (1,)), ((), ())),
                        preferred_element_type=jnp.float32,
                        precision=lax.Precision.HIGHEST) + c1_ref[...]
    h = jnp.clip(h, 0.0, _CLAMP)
    h = lax.dot_general(h, w2_ref[...], (((1,), (1,)), ((), ())),
                        preferred_element_type=jnp.float32,
                        precision=lax.Precision.HIGHEST) + c2_ref[...]
    h = jnp.clip(h, 0.0, _CLAMP)
    yv = jnp.sum(h * w3_ref[...], axis=1, keepdims=True) + c3_ref[0, 0]
    y_ref[...] = yv


def _mlp(acc_w, acc_b, stm, b1, fc1_w, fc1_b, fc2_w, fc2_b, out_w, out_b):
    grid = (_B // _BLK,)
    return pl.pallas_call(
        _mlp_body,
        grid=grid,
        in_specs=[
            pl.BlockSpec((_BLK, _H), lambda i: (i, 0)),
            pl.BlockSpec((_BLK, _H), lambda i: (i, 0)),
            pl.BlockSpec((_BLK, 1), lambda i: (i, 0)),
            pl.BlockSpec((1, _H), lambda i: (0, 0)),
            pl.BlockSpec((32, 2 * _H), lambda i: (0, 0)),
            pl.BlockSpec((1, 32), lambda i: (0, 0)),
            pl.BlockSpec((32, 32), lambda i: (0, 0)),
            pl.BlockSpec((1, 32), lambda i: (0, 0)),
            pl.BlockSpec((1, 32), lambda i: (0, 0)),
            pl.BlockSpec((1, 1), lambda i: (0, 0)),
        ],
        out_specs=pl.BlockSpec((_BLK, 1), lambda i: (i, 0)),
        out_shape=jax.ShapeDtypeStruct((_B, 1), jnp.float32),
    )(acc_w, acc_b, stm, b1.reshape(1, _H),
      fc1_w, fc1_b.reshape(1, 32), fc2_w, fc2_b.reshape(1, 32), out_w,
      out_b.reshape(1, 1))[:, 0]


def kernel(feats_w, feats_b, stm, emb, b1, fc1_w, fc1_b, fc2_w, fc2_b,
           out_w, out_b):
    feats = jnp.concatenate([feats_w, feats_b], axis=0)
    idx = feats.reshape(_NW * _NCH, _CBAG, _L).transpose(0, 2, 1)
    idx = idx.astype(jnp.int32)
    acc = _bag(idx, emb)
    return _mlp(acc[:_B], acc[_B:], stm.reshape(_B, 1), b1,
                fc1_w, fc1_b, fc2_w, fc2_b, out_w, out_b)


# speedup vs baseline: 1.3085x; 1.3085x over previous
import jax
import jax.numpy as jnp
from jax import lax
from jax.experimental import pallas as pl
from jax.experimental.pallas import tpu as pltpu
from jax.experimental.pallas import tpu_sc as plsc

_B, _L, _V, _H = 16384, 50, 100000, 256
_CLAMP = 127.0

_NC, _NS = 2, 16
_NW = _NC * _NS
_NBAG = 2 * _B
_BPW = _NBAG // _NW
_CBAG = 128
_NCH = _BPW // _CBAG
_NVR = _H // 16
_NSPL = 4


def _bag_body(idx_ref, emb_ref, out_ref, idx_v, buf_v, acc_v, sem0, sem1):
    cid = lax.axis_index("c")
    sid = lax.axis_index("s")
    wid = sid * _NC + cid
    zero = jnp.zeros((16,), jnp.float32)

    def chunk(c, carry):
        pltpu.sync_copy(idx_ref.at[wid * _NCH + c], idx_v)

        @plsc.parallel_loop(0, _CBAG, unroll=2)
        def _(r):
            for j in range(_NVR):
                acc_v[r, pl.ds(16 * j, 16)] = zero

        mhi = jnp.int32(-65536)

        def accum_pair(k):
            @plsc.parallel_loop(0, _CBAG, unroll=2)
            def _(r):
                for j in range(_NVR // 2):
                    v0 = buf_v[2 * k, r, pl.ds(16 * j, 16)]
                    v1 = buf_v[2 * k + 1, r, pl.ds(16 * j, 16)]
                    a = (lax.bitcast_convert_type(v0 << 16, jnp.float32)
                         + lax.bitcast_convert_type(v1 << 16, jnp.float32))
                    b = (lax.bitcast_convert_type(v0 & mhi, jnp.float32)
                         + lax.bitcast_convert_type(v1 & mhi, jnp.float32))
                    plsc.addupdate(acc_v.at[r, pl.ds(32 * j, 16)], a)
                    plsc.addupdate(acc_v.at[r, pl.ds(32 * j + 16, 16)], b)

        def gath(q, k, sem):
            pltpu.async_copy(emb_ref.at[idx_v.at[2 * q]],
                             buf_v.at[2 * k], sem)
            pltpu.async_copy(emb_ref.at[idx_v.at[2 * q + 1]],
                             buf_v.at[2 * k + 1], sem)

        def waitg(q, k, sem):
            pltpu.make_async_copy(emb_ref.at[idx_v.at[2 * q]],
                                  buf_v.at[2 * k], sem).wait()
            pltpu.make_async_copy(emb_ref.at[idx_v.at[2 * q + 1]],
                                  buf_v.at[2 * k + 1], sem).wait()

        _NQ = _L // 2
        gath(0, 0, sem0)

        def pos(i, cc):
            q0 = 2 * i

            @pl.when(q0 + 1 < _NQ)
            def _():
                gath(q0 + 1, 1, sem1)

            waitg(q0, 0, sem0)
            accum_pair(0)

            @pl.when(q0 + 2 < _NQ)
            def _():
                gath(q0 + 2, 0, sem0)

            @pl.when(q0 + 1 < _NQ)
            def _():
                waitg(q0 + 1, 1, sem1)
                accum_pair(1)

            return cc

        lax.fori_loop(0, (_NQ + 1) // 2, pos, 0)
        pltpu.sync_copy(acc_v,
                        out_ref.at[pl.ds(wid * _BPW + c * _CBAG, _CBAG), :])
        return carry

    lax.fori_loop(0, _NCH, chunk, 0)


_bag = pl.kernel(
    _bag_body,
    out_type=jax.ShapeDtypeStruct((_NBAG, _H), jnp.float32),
    mesh=plsc.VectorSubcoreMesh(
        core_axis_name="c", subcore_axis_name="s",
        num_cores=_NC, num_subcores=_NS,
    ),
    scratch_types=[
        pltpu.VMEM((_L, _CBAG), jnp.int32),
        pltpu.VMEM((4, _CBAG, _H // 2), jnp.int32),
        pltpu.VMEM((_CBAG, _H), jnp.float32),
        pltpu.SemaphoreType.DMA,
        pltpu.SemaphoreType.DMA,
    ],
)

_BLK = 2048


def _mlp_body(accw_ref, accb_ref, stm_ref, b1_ref, w1_ref, c1_ref,
              w2_ref, c2_ref, w3_ref, c3_ref, y_ref):
    white = stm_ref[...] == 0
    aw = accw_ref[...] + b1_ref[...]
    ab = accb_ref[...] + b1_ref[...]
    astm = jnp.where(white, aw, ab)
    anstm = jnp.where(white, ab, aw)
    x = jnp.clip(jnp.concatenate([astm, anstm], axis=1), 0.0, _CLAMP)
    h = lax.dot_general(x, w1_ref[...], (((1,), (1,)), ((), ())),
                        preferred_element_type=jnp.float32,
                        precision=lax.Precision.HIGHEST) + c1_ref[...]
    h = jnp.clip(h, 0.0, _CLAMP)
    h = lax.dot_general(h, w2_ref[...], (((1,), (1,)), ((), ())),
                        preferred_element_type=jnp.float32,
                        precision=lax.Precision.HIGHEST) + c2_ref[...]
    h = jnp.clip(h, 0.0, _CLAMP)
    yv = jnp.sum(h * w3_ref[...], axis=1, keepdims=True) + c3_ref[0, 0]
    y_ref[...] = yv


def _mlp(acc_w, acc_b, stm, b1, fc1_w, fc1_b, fc2_w, fc2_b, out_w, out_b):
    grid = (_B // _BLK,)
    return pl.pallas_call(
        _mlp_body,
        grid=grid,
        in_specs=[
            pl.BlockSpec((_BLK, _H), lambda i: (i, 0)),
            pl.BlockSpec((_BLK, _H), lambda i: (i, 0)),
            pl.BlockSpec((_BLK, 1), lambda i: (i, 0)),
            pl.BlockSpec((1, _H), lambda i: (0, 0)),
            pl.BlockSpec((32, 2 * _H), lambda i: (0, 0)),
            pl.BlockSpec((1, 32), lambda i: (0, 0)),
            pl.BlockSpec((32, 32), lambda i: (0, 0)),
            pl.BlockSpec((1, 32), lambda i: (0, 0)),
            pl.BlockSpec((1, 32), lambda i: (0, 0)),
            pl.BlockSpec((1, 1), lambda i: (0, 0)),
        ],
        out_specs=pl.BlockSpec((_BLK, 1), lambda i: (i, 0)),
        out_shape=jax.ShapeDtypeStruct((_B, 1), jnp.float32),
    )(acc_w, acc_b, stm, b1.reshape(1, _H),
      fc1_w, fc1_b.reshape(1, 32), fc2_w, fc2_b.reshape(1, 32), out_w,
      out_b.reshape(1, 1))[:, 0]


def kernel(feats_w, feats_b, stm, emb, b1, fc1_w, fc1_b, fc2_w, fc2_b,
           out_w, out_b):
    feats = jnp.concatenate([feats_w, feats_b], axis=0)
    idx = feats.reshape(_NW * _NCH, _CBAG, _L).transpose(0, 2, 1)
    idx = idx.astype(jnp.int32)
    emb_bf = emb.astype(jnp.bfloat16)
    emb_bf = emb_bf.reshape(_V + 1, _H // 32, 2, 16)
    emb_bf = emb_bf.transpose(0, 1, 3, 2).reshape(_V + 1, _H // 2, 2)
    emb_i32 = lax.bitcast_convert_type(emb_bf, jnp.int32)
    acc = _bag(idx, emb_i32)
    return _mlp(acc[:_B], acc[_B:], stm.reshape(_B, 1), b1,
                fc1_w, fc1_b, fc2_w, fc2_b, out_w, out_b)
